# baseline (speedup 1.0000x reference)
import functools

import jax
import jax.numpy as jnp
from jax import lax
from jax.experimental import pallas as pl
from jax.experimental.pallas import tpu as pltpu
from jax.experimental.pallas import tpu_sc as plsc

ENTRIES = 100000
DIM2 = 128
BATCH = 4096
HIST = 50

NC = 2
NS = 16
NW = NC * NS
CH = BATCH // NW
NCH = HIST
NBUF = 5
PRE = 3

_mesh = plsc.VectorSubcoreMesh(core_axis_name="c", subcore_axis_name="s")


@functools.partial(
    pl.kernel,
    out_type=jax.ShapeDtypeStruct((HIST, BATCH, DIM2), jnp.float32),
    mesh=_mesh,
    scratch_types=[
        pltpu.VMEM((NCH, CH), jnp.int32),
        [pltpu.VMEM((CH, DIM2), jnp.float32)] * NBUF,
        [pltpu.SemaphoreType.DMA] * NBUF,
        [pltpu.SemaphoreType.DMA] * NBUF,
    ],
)
def _gather(idx_hbm, table_hbm, out_hbm, idx_v, rows, gsem, wsem):
    wid = lax.axis_index("s") * NC + lax.axis_index("c")
    base = wid * CH
    pltpu.sync_copy(idx_hbm.at[:, pl.ds(base, CH)], idx_v)

    def out_slice(h):
        return out_hbm.at[h, pl.ds(base, CH)]

    def issue_g(h, b):
        pltpu.async_copy(table_hbm.at[idx_v.at[h]], rows[b], gsem[b])

    def wait_g(h, b):
        pltpu.make_async_copy(table_hbm.at[idx_v.at[h]], rows[b],
                              gsem[b]).wait()

    def issue_w(h, b):
        pltpu.async_copy(rows[b], out_slice(h), wsem[b])

    def wait_w(h, b):
        pltpu.make_async_copy(rows[b], out_slice(h), wsem[b]).wait()

    for b in range(PRE):
        issue_g(b, b)

    for s in range(NBUF):
        wait_g(s, s)
        issue_w(s, s)
        bi = (s + PRE) % NBUF
        if s >= NBUF - PRE:
            wait_w(s - (NBUF - PRE), bi)
        issue_g(s + PRE, bi)

    def round_(i, carry):
        g = i * NBUF + NBUF
        for s in range(NBUF):
            h = g + s
            b = s
            wait_g(h, b)
            issue_w(h, b)
            bi = (s + PRE) % NBUF
            wait_w(h - (NBUF - PRE), bi)
            issue_g(h + PRE, bi)
        return carry

    lax.fori_loop(0, NCH // NBUF - 2, round_, 0)

    for h in range(NCH - NBUF, NCH):
        b = h % NBUF
        wait_g(h, b)
        issue_w(h, b)
        if h + PRE < NCH:
            bi = (h + PRE) % NBUF
            wait_w(h - (NBUF - PRE), bi)
            issue_g(h + PRE, bi)

    for h in range(NCH - NBUF, NCH):
        wait_w(h, h % NBUF)


def kernel(x, weight):
    out = _gather(x.T.astype(jnp.int32), weight)
    return jnp.transpose(out, (1, 0, 2))


# speedup vs baseline: 1.0059x; 1.0059x over previous
import functools

import jax
import jax.numpy as jnp
from jax import lax
from jax.experimental import pallas as pl
from jax.experimental.pallas import tpu as pltpu
from jax.experimental.pallas import tpu_sc as plsc

ENTRIES = 100000
DIM2 = 128
BATCH = 4096
HIST = 50

NC = 2
NS = 16
NW = NC * NS
CH = BATCH // NW
NCH = HIST
NBUF = 5

_mesh = plsc.VectorSubcoreMesh(core_axis_name="c", subcore_axis_name="s")


@functools.partial(
    pl.kernel,
    out_type=jax.ShapeDtypeStruct((HIST, BATCH, DIM2), jnp.float32),
    mesh=_mesh,
    scratch_types=[
        pltpu.VMEM((NCH, CH), jnp.int32),
        [pltpu.VMEM((CH, DIM2), jnp.float32)] * NBUF,
        [pltpu.SemaphoreType.DMA] * NBUF,
        [pltpu.SemaphoreType.DMA] * NBUF,
    ],
    compiler_params=pltpu.CompilerParams(skip_device_barrier=True),
)
def _gather(idx_hbm, table_hbm, out_hbm, idx_v, rows, gsem, wsem):
    wid = lax.axis_index("s") * NC + lax.axis_index("c")
    base = wid * CH
    pltpu.sync_copy(idx_hbm.at[:, pl.ds(base, CH)], idx_v)

    def out_slice(h):
        return out_hbm.at[h, pl.ds(base, CH)]

    for b in range(NBUF):
        pltpu.async_copy(table_hbm.at[idx_v.at[b]], rows[b], gsem[b])

    def round_(i, carry):
        g = i * NBUF
        for b in range(NBUF):
            h = g + b
            pltpu.make_async_copy(table_hbm.at[idx_v.at[h]], rows[b],
                                  gsem[b]).wait()
            pltpu.async_copy(rows[b], out_slice(h), wsem[b]).wait()
            pltpu.async_copy(table_hbm.at[idx_v.at[h + NBUF]], rows[b],
                             gsem[b])
        return carry

    lax.fori_loop(0, NCH // NBUF - 1, round_, 0)

    for b in range(NBUF):
        h = NCH - NBUF + b
        pltpu.make_async_copy(table_hbm.at[idx_v.at[h]], rows[b],
                              gsem[b]).wait()
        pltpu.sync_copy(rows[b], out_slice(h))


def kernel(x, weight):
    out = _gather(x.T.astype(jnp.int32), weight)
    return jnp.transpose(out, (1, 0, 2))
